# initial kernel scaffold (unmeasured)
import jax
import jax.numpy as jnp
from jax import lax
from jax.experimental import pallas as pl
from jax.experimental.pallas import tpu as pltpu

N_DEV = 32


def kernel(x, w_mat):
    m, k_local = x.shape
    _, n = w_mat.shape
    chunk = m // N_DEV

    def body(x_ref, w_ref, out_ref, send_buf, recv_buf,
             send_sem, recv_sem, credit_sem):
        me = lax.axis_index("i")
        left = lax.rem(me + N_DEV - 1, N_DEV)
        right = lax.rem(me + 1, N_DEV)

        barrier_sem = pltpu.get_barrier_semaphore()
        pl.semaphore_signal(barrier_sem, inc=1, device_id=(left,),
                            device_id_type=pl.DeviceIdType.MESH)
        pl.semaphore_signal(barrier_sem, inc=1, device_id=(right,),
                            device_id_type=pl.DeviceIdType.MESH)
        pl.semaphore_wait(barrier_sem, 2)

        out_ref[:, :] = jnp.dot(
            x_ref[:, :].astype(jnp.bfloat16),
            w_ref[:, :].astype(jnp.bfloat16),
            preferred_element_type=jnp.float32,
        )

        def cs(c):
            return pl.ds(c * chunk, chunk)

        def ring_send(step_idx):
            @pl.when(step_idx > 0)
            def _():
                pl.semaphore_wait(credit_sem, 1)
            rdma = pltpu.make_async_remote_copy(
                src_ref=send_buf,
                dst_ref=recv_buf,
                send_sem=send_sem,
                recv_sem=recv_sem,
                device_id=(right,),
                device_id_type=pl.DeviceIdType.MESH,
            )
            rdma.start()
            rdma.wait()

        def credit_left(step_idx):
            @pl.when(step_idx < 2 * (N_DEV - 1) - 1)
            def _():
                pl.semaphore_signal(credit_sem, inc=1, device_id=(left,),
                                    device_id_type=pl.DeviceIdType.MESH)

        def rs_step(s, carry):
            src_c = lax.rem(me - s + 2 * N_DEV, N_DEV)
            dst_c = lax.rem(me - s - 1 + 2 * N_DEV, N_DEV)
            send_buf[:, :] = out_ref[cs(src_c), :].astype(jnp.bfloat16)
            ring_send(s)
            out_ref[cs(dst_c), :] += recv_buf[:, :].astype(jnp.float32)
            credit_left(s)
            return carry

        lax.fori_loop(0, N_DEV - 1, rs_step, 0)

        own = lax.rem(me + 1, N_DEV)
        v = out_ref[cs(own), :]
        out_ref[cs(own), :] = v * jax.nn.sigmoid(v)

        def ag_step(t, carry):
            src_c = lax.rem(own - t + 2 * N_DEV, N_DEV)
            dst_c = lax.rem(own - t - 1 + 2 * N_DEV, N_DEV)
            send_buf[:, :] = out_ref[cs(src_c), :].astype(jnp.bfloat16)
            ring_send(t + N_DEV - 1)
            out_ref[cs(dst_c), :] = recv_buf[:, :].astype(jnp.float32)
            credit_left(t + N_DEV - 1)
            return carry

        lax.fori_loop(0, N_DEV - 1, ag_step, 0)

    return pl.pallas_call(
        body,
        out_shape=jax.ShapeDtypeStruct((m, n), jnp.float32),
        in_specs=[
            pl.BlockSpec(memory_space=pltpu.VMEM),
            pl.BlockSpec(memory_space=pltpu.VMEM),
        ],
        out_specs=pl.BlockSpec(memory_space=pltpu.VMEM),
        scratch_shapes=[
            pltpu.VMEM((chunk, n), jnp.bfloat16),
            pltpu.VMEM((chunk, n), jnp.bfloat16),
            pltpu.SemaphoreType.DMA,
            pltpu.SemaphoreType.DMA,
            pltpu.SemaphoreType.REGULAR,
        ],
        compiler_params=pltpu.CompilerParams(collective_id=0),
    )(x, w_mat)


# baseline (device time: 834363 ns/iter reference)
import jax
import jax.numpy as jnp
from jax import lax
from jax.experimental import pallas as pl
from jax.experimental.pallas import tpu as pltpu

N_DEV = 32


def kernel(x, w_mat):
    m, k_local = x.shape
    _, n = w_mat.shape
    chunk = m // N_DEV

    def body(x_ref, w_ref, out_ref, send_buf, recv_buf,
             send_sem, recv_sem, credit_sem):
        me = lax.axis_index("i")
        left = lax.rem(me + N_DEV - 1, N_DEV)
        right = lax.rem(me + 1, N_DEV)

        barrier_sem = pltpu.get_barrier_semaphore()
        pl.semaphore_signal(barrier_sem, inc=1, device_id=(left,),
                            device_id_type=pl.DeviceIdType.MESH)
        pl.semaphore_signal(barrier_sem, inc=1, device_id=(right,),
                            device_id_type=pl.DeviceIdType.MESH)
        pl.semaphore_wait(barrier_sem, 2)

        out_ref[:, :] = jnp.dot(
            x_ref[:, :].astype(jnp.bfloat16),
            w_ref[:, :].astype(jnp.bfloat16),
            preferred_element_type=jnp.float32,
        )

        def cs(c):
            return pl.ds(c * chunk, chunk)

        def ring_send(step_idx):
            @pl.when(step_idx > 0)
            def _():
                pl.semaphore_wait(credit_sem, 1)
            rdma = pltpu.make_async_remote_copy(
                src_ref=send_buf,
                dst_ref=recv_buf,
                send_sem=send_sem,
                recv_sem=recv_sem,
                device_id=(right,),
                device_id_type=pl.DeviceIdType.MESH,
            )
            rdma.start()
            rdma.wait()

        def credit_left(step_idx):
            @pl.when(step_idx < 2 * (N_DEV - 1) - 1)
            def _():
                pl.semaphore_signal(credit_sem, inc=1, device_id=(left,),
                                    device_id_type=pl.DeviceIdType.MESH)

        def rs_step(s, carry):
            src_c = lax.rem(me - s + 2 * N_DEV, N_DEV)
            dst_c = lax.rem(me - s - 1 + 2 * N_DEV, N_DEV)
            send_buf[:, :] = out_ref[cs(src_c), :].astype(jnp.bfloat16)
            ring_send(s)
            out_ref[cs(dst_c), :] += recv_buf[:, :].astype(jnp.float32)
            credit_left(s)
            return carry

        lax.fori_loop(0, N_DEV - 1, rs_step, 0)

        own = lax.rem(me + 1, N_DEV)
        v = out_ref[cs(own), :]
        out_ref[cs(own), :] = v * jax.nn.sigmoid(v)

        def ag_step(t, carry):
            src_c = lax.rem(own - t + 2 * N_DEV, N_DEV)
            dst_c = lax.rem(own - t - 1 + 2 * N_DEV, N_DEV)
            send_buf[:, :] = out_ref[cs(src_c), :].astype(jnp.bfloat16)
            ring_send(t + N_DEV - 1)
            out_ref[cs(dst_c), :] = recv_buf[:, :].astype(jnp.float32)
            credit_left(t + N_DEV - 1)
            return carry

        lax.fori_loop(0, N_DEV - 1, ag_step, 0)

    return pl.pallas_call(
        body,
        out_shape=jax.ShapeDtypeStruct((m, n), jnp.float32),
        in_specs=[
            pl.BlockSpec(memory_space=pltpu.VMEM),
            pl.BlockSpec(memory_space=pltpu.VMEM),
        ],
        out_specs=pl.BlockSpec(memory_space=pltpu.VMEM),
        scratch_shapes=[
            pltpu.VMEM((chunk, n), jnp.bfloat16),
            pltpu.VMEM((chunk, n), jnp.bfloat16),
            pltpu.SemaphoreType.DMA,
            pltpu.SemaphoreType.DMA,
            pltpu.SemaphoreType.REGULAR,
        ],
        compiler_params=pltpu.CompilerParams(
            collective_id=0,
            vmem_limit_bytes=100 * 1024 * 1024,
        ),
    )(x, w_mat)


# device time: 375128 ns/iter; 2.2242x vs baseline; 2.2242x over previous
import jax
import jax.numpy as jnp
from jax import lax
from jax.experimental import pallas as pl
from jax.experimental.pallas import tpu as pltpu

N_DEV = 32

RING = [0, 1, 2, 3, 4, 5, 6, 7, 15, 14, 13, 12, 11, 10, 9, 17,
        18, 19, 20, 21, 22, 23, 31, 30, 29, 28, 27, 26, 25, 24, 16, 8]
POS = [0] * N_DEV
for _p, _d in enumerate(RING):
    POS[_d] = _p
RIGHT = [RING[(POS[d] + 1) % N_DEV] for d in range(N_DEV)]
LEFT = [RING[(POS[d] - 1) % N_DEV] for d in range(N_DEV)]


def kernel(x, w_mat):
    m, k_local = x.shape
    _, n = w_mat.shape
    chunk = m // N_DEV
    half = n // 2

    def body(x_ref, w_ref, tbl_ref, out_ref, send_buf, recv_buf,
             send_sems, recv_sems, credit_f, credit_b):
        me = lax.axis_index("i")

        lane = lax.broadcasted_iota(jnp.int32, (1, 128), 1)

        def lut(row):
            return jnp.sum(jnp.where(lane == me, tbl_ref[row:row + 1, :], 0))

        r = lut(0)
        right_id = lut(1)
        left_id = lut(2)
        rb = lax.rem(N_DEV - r, N_DEV)

        barrier_sem = pltpu.get_barrier_semaphore()
        pl.semaphore_signal(barrier_sem, inc=1, device_id=(left_id,),
                            device_id_type=pl.DeviceIdType.MESH)
        pl.semaphore_signal(barrier_sem, inc=1, device_id=(right_id,),
                            device_id_type=pl.DeviceIdType.MESH)
        pl.semaphore_wait(barrier_sem, 2)

        out_ref[:, :] = jnp.dot(
            x_ref[:, :].astype(jnp.bfloat16),
            w_ref[:, :].astype(jnp.bfloat16),
            preferred_element_type=jnp.float32,
        )

        def cs(c):
            return pl.ds(c * chunk, chunk)

        def ring_exchange(step_idx):
            @pl.when(step_idx > 0)
            def _():
                pl.semaphore_wait(credit_f, 1)
                pl.semaphore_wait(credit_b, 1)
            rdma_f = pltpu.make_async_remote_copy(
                src_ref=send_buf.at[0], dst_ref=recv_buf.at[0],
                send_sem=send_sems.at[0], recv_sem=recv_sems.at[0],
                device_id=(right_id,), device_id_type=pl.DeviceIdType.MESH,
            )
            rdma_b = pltpu.make_async_remote_copy(
                src_ref=send_buf.at[1], dst_ref=recv_buf.at[1],
                send_sem=send_sems.at[1], recv_sem=recv_sems.at[1],
                device_id=(left_id,), device_id_type=pl.DeviceIdType.MESH,
            )
            rdma_f.start()
            rdma_b.start()
            rdma_f.wait()
            rdma_b.wait()

        def credit_senders(step_idx):
            @pl.when(step_idx < 2 * (N_DEV - 1) - 1)
            def _():
                pl.semaphore_signal(credit_f, inc=1, device_id=(left_id,),
                                    device_id_type=pl.DeviceIdType.MESH)
                pl.semaphore_signal(credit_b, inc=1, device_id=(right_id,),
                                    device_id_type=pl.DeviceIdType.MESH)

        def rs_step(s, carry):
            src_f = lax.rem(r - s + 2 * N_DEV, N_DEV)
            dst_f = lax.rem(r - s - 1 + 2 * N_DEV, N_DEV)
            src_b = lax.rem(rb - s + 2 * N_DEV, N_DEV)
            dst_b = lax.rem(rb - s - 1 + 2 * N_DEV, N_DEV)
            send_buf[0] = out_ref[cs(src_f), :half].astype(jnp.bfloat16)
            send_buf[1] = out_ref[cs(src_b), half:].astype(jnp.bfloat16)
            ring_exchange(s)
            out_ref[cs(dst_f), :half] += recv_buf[0].astype(jnp.float32)
            out_ref[cs(dst_b), half:] += recv_buf[1].astype(jnp.float32)
            credit_senders(s)
            return carry

        lax.fori_loop(0, N_DEV - 1, rs_step, 0)

        own_f = lax.rem(r + 1, N_DEV)
        own_b = lax.rem(rb + 1, N_DEV)
        vf = out_ref[cs(own_f), :half]
        out_ref[cs(own_f), :half] = vf * jax.nn.sigmoid(vf)
        vb = out_ref[cs(own_b), half:]
        out_ref[cs(own_b), half:] = vb * jax.nn.sigmoid(vb)

        def ag_step(t, carry):
            src_f = lax.rem(own_f - t + 2 * N_DEV, N_DEV)
            dst_f = lax.rem(own_f - t - 1 + 2 * N_DEV, N_DEV)
            src_b = lax.rem(own_b - t + 2 * N_DEV, N_DEV)
            dst_b = lax.rem(own_b - t - 1 + 2 * N_DEV, N_DEV)
            send_buf[0] = out_ref[cs(src_f), :half].astype(jnp.bfloat16)
            send_buf[1] = out_ref[cs(src_b), half:].astype(jnp.bfloat16)
            ring_exchange(t + N_DEV - 1)
            out_ref[cs(dst_f), :half] = recv_buf[0].astype(jnp.float32)
            out_ref[cs(dst_b), half:] = recv_buf[1].astype(jnp.float32)
            credit_senders(t + N_DEV - 1)
            return carry

        lax.fori_loop(0, N_DEV - 1, ag_step, 0)

    pad = [0] * (128 - N_DEV)
    tables = jnp.array(
        [POS + pad, RIGHT + pad, LEFT + pad], dtype=jnp.int32
    )

    return pl.pallas_call(
        body,
        out_shape=jax.ShapeDtypeStruct((m, n), jnp.float32),
        in_specs=[
            pl.BlockSpec(memory_space=pltpu.VMEM),
            pl.BlockSpec(memory_space=pltpu.VMEM),
            pl.BlockSpec(memory_space=pltpu.VMEM),
        ],
        out_specs=pl.BlockSpec(memory_space=pltpu.VMEM),
        scratch_shapes=[
            pltpu.VMEM((2, chunk, n // 2), jnp.bfloat16),
            pltpu.VMEM((2, chunk, n // 2), jnp.bfloat16),
            pltpu.SemaphoreType.DMA((2,)),
            pltpu.SemaphoreType.DMA((2,)),
            pltpu.SemaphoreType.REGULAR,
            pltpu.SemaphoreType.REGULAR,
        ],
        compiler_params=pltpu.CompilerParams(
            collective_id=0,
            vmem_limit_bytes=100 * 1024 * 1024,
        ),
    )(x, w_mat, tables)


# device time: 329849 ns/iter; 2.5295x vs baseline; 1.1373x over previous
import jax
import jax.numpy as jnp
from jax import lax
from jax.experimental import pallas as pl
from jax.experimental.pallas import tpu as pltpu

N_DEV = 32
N_SUB = 4
N_STREAM = 2 * N_SUB
N_SLOT = 4

RING = [0, 1, 2, 3, 4, 5, 6, 7, 15, 14, 13, 12, 11, 10, 9, 17,
        18, 19, 20, 21, 22, 23, 31, 30, 29, 28, 27, 26, 25, 24, 16, 8]
POS = [0] * N_DEV
for _p, _d in enumerate(RING):
    POS[_d] = _p
RIGHT = [RING[(POS[d] + 1) % N_DEV] for d in range(N_DEV)]
LEFT = [RING[(POS[d] - 1) % N_DEV] for d in range(N_DEV)]


def kernel(x, w_mat):
    m, k_local = x.shape
    _, n = w_mat.shape
    chunk = m // N_DEV
    sub = n // N_STREAM

    def body(x_ref, w_ref, tbl_ref, out_ref, send_buf, recv_buf,
             send_sems, recv_sems, *credit_sems):
        me = lax.axis_index("i")

        lane = lax.broadcasted_iota(jnp.int32, (1, 128), 1)

        def lut(row):
            return jnp.sum(jnp.where(lane == me, tbl_ref[row:row + 1, :], 0))

        r = lut(0)
        right_id = lut(1)
        left_id = lut(2)
        rb = lax.rem(N_DEV - r, N_DEV)

        pos = [r] * N_SUB + [rb] * N_SUB
        own = [lax.rem(p + 1, N_DEV) for p in pos]
        send_peer = [right_id] * N_SUB + [left_id] * N_SUB
        credit_peer = [left_id] * N_SUB + [right_id] * N_SUB
        rsub = chunk // N_STREAM
        roff = [j * rsub for j in range(N_STREAM)]

        barrier_sem = pltpu.get_barrier_semaphore()
        pl.semaphore_signal(barrier_sem, inc=1, device_id=(left_id,),
                            device_id_type=pl.DeviceIdType.MESH)
        pl.semaphore_signal(barrier_sem, inc=1, device_id=(right_id,),
                            device_id_type=pl.DeviceIdType.MESH)
        pl.semaphore_wait(barrier_sem, 2)

        out_ref[:, :] = jnp.dot(
            x_ref[:, :].astype(jnp.bfloat16),
            w_ref[:, :].astype(jnp.bfloat16),
            preferred_element_type=jnp.float32,
        )

        def oref(j, c):
            return out_ref[pl.ds(c * chunk + roff[j], rsub), :]

        def set_oref(j, c, val):
            out_ref[pl.ds(c * chunk + roff[j], rsub), :] = val

        def start_send(j, src, slot4, slot2):
            pltpu.make_async_remote_copy(
                src_ref=src,
                dst_ref=recv_buf.at[j, slot4],
                send_sem=send_sems.at[j, slot2],
                recv_sem=recv_sems.at[j, slot4],
                device_id=(send_peer[j],),
                device_id_type=pl.DeviceIdType.MESH,
            ).start()

        def wait_send_slot(j, slot2):
            pltpu.make_async_remote_copy(
                src_ref=send_buf.at[j, 0],
                dst_ref=recv_buf.at[j, 0],
                send_sem=send_sems.at[j, slot2],
                recv_sem=recv_sems.at[j, 0],
                device_id=(send_peer[j],),
                device_id_type=pl.DeviceIdType.MESH,
            ).wait_send()

        def wait_recv_slot(j, slot4):
            pltpu.make_async_remote_copy(
                src_ref=send_buf.at[j, 0],
                dst_ref=recv_buf.at[j, slot4],
                send_sem=send_sems.at[j, 0],
                recv_sem=recv_sems.at[j, slot4],
                device_id=(send_peer[j],),
                device_id_type=pl.DeviceIdType.MESH,
            ).wait_recv()

        def send_credit(j):
            pl.semaphore_signal(credit_sems[j], inc=1,
                                device_id=(credit_peer[j],),
                                device_id_type=pl.DeviceIdType.MESH)

        for j in range(N_STREAM):
            send_buf[j, 0] = oref(j, pos[j]).astype(jnp.bfloat16)

        def rs_step(s, k, credit_wait, wait_prev, last):
            slot4 = k % 4
            slot2 = k % 2
            for j in range(N_STREAM):
                if credit_wait:
                    pl.semaphore_wait(credit_sems[j], 1)
                start_send(j, send_buf.at[j, slot2], slot4, slot2)
                if wait_prev:
                    wait_send_slot(j, (slot2 + 1) % 2)
            for j in range(N_STREAM):
                wait_recv_slot(j, slot4)
                dst = lax.rem(pos[j] - s - 1 + 2 * N_DEV, N_DEV)
                if not last:
                    send_buf[j, (k + 1) % 2] = (
                        oref(j, dst)
                        + recv_buf[j, slot4].astype(jnp.float32)
                    ).astype(jnp.bfloat16)
                else:
                    full = (oref(j, dst)
                            + recv_buf[j, slot4].astype(jnp.float32))
                    y = full * jax.nn.sigmoid(full)
                    set_oref(j, dst, y)
                    send_buf[j, (k + 1) % 2] = y.astype(jnp.bfloat16)
                send_credit(j)

        rs_step(0, 0, False, False, False)
        for k in (1, 2, 3):
            rs_step(k, k, False, True, False)

        def rs_block(b, carry):
            s = 4 * b
            for k in range(4):
                rs_step(s + k, k, True, True, False)
            return carry

        lax.fori_loop(1, 7, rs_block, 0)

        rs_step(28, 0, True, True, False)
        rs_step(29, 1, True, True, False)
        rs_step(30, 2, True, True, True)

        def ag_step(t, slot4, slot2, prev_slot4, first, signal, guard, store):
            for j in range(N_STREAM):
                pl.semaphore_wait(credit_sems[j], 1)
                if first:
                    start_send(j, send_buf.at[j, slot2], slot4, slot2)
                else:
                    start_send(j, recv_buf.at[j, prev_slot4], slot4, slot2)
                wait_send_slot(j, (slot2 + 1) % 2)
                if signal:
                    if guard:
                        @pl.when(t <= 28)
                        def _(j=j):
                            send_credit(j)
                    else:
                        send_credit(j)
            for j in range(N_STREAM):
                if store:
                    c = lax.rem(own[j] - t + 2 * N_DEV, N_DEV)
                    set_oref(j, c,
                             recv_buf[j, prev_slot4].astype(jnp.float32))
                wait_recv_slot(j, slot4)

        ag_step(0, 3, 1, None, True, False, False, False)
        ag_step(1, 0, 0, 3, False, False, False, True)
        ag_step(2, 1, 1, 0, False, True, False, True)

        def ag_block(b, carry):
            t0 = 3 + 4 * b
            for k in range(4):
                ag_step(t0 + k, (2 + k) % 4, k % 2, (1 + k) % 4,
                        False, True, True, True)
            return carry

        lax.fori_loop(0, 7, ag_block, 0)

        for j in range(N_STREAM):
            c = lax.rem(own[j] + 1, N_DEV)
            set_oref(j, c, recv_buf[j, 1].astype(jnp.float32))
            wait_send_slot(j, 1)

    pad = [0] * (128 - N_DEV)
    tables = jnp.array(
        [POS + pad, RIGHT + pad, LEFT + pad], dtype=jnp.int32
    )

    return pl.pallas_call(
        body,
        out_shape=jax.ShapeDtypeStruct((m, n), jnp.float32),
        in_specs=[
            pl.BlockSpec(memory_space=pltpu.VMEM),
            pl.BlockSpec(memory_space=pltpu.VMEM),
            pl.BlockSpec(memory_space=pltpu.VMEM),
        ],
        out_specs=pl.BlockSpec(memory_space=pltpu.VMEM),
        scratch_shapes=[
            pltpu.VMEM((N_STREAM, 2, chunk // N_STREAM, n), jnp.bfloat16),
            pltpu.VMEM((N_STREAM, N_SLOT, chunk // N_STREAM, n), jnp.bfloat16),
            pltpu.SemaphoreType.DMA((N_STREAM, 2)),
            pltpu.SemaphoreType.DMA((N_STREAM, N_SLOT)),
        ] + [pltpu.SemaphoreType.REGULAR] * N_STREAM,
        compiler_params=pltpu.CompilerParams(
            collective_id=0,
            vmem_limit_bytes=100 * 1024 * 1024,
        ),
    )(x, w_mat, tables)


# device time: 222054 ns/iter; 3.7575x vs baseline; 1.4854x over previous
import jax
import jax.numpy as jnp
from jax import lax
from jax.experimental import pallas as pl
from jax.experimental.pallas import tpu as pltpu

N_DEV = 32
N_SUB = 4
N_STREAM = 2 * N_SUB
N_SLOT = 4

RING = [0, 1, 2, 3, 4, 5, 6, 7, 15, 14, 13, 12, 11, 10, 9, 17,
        18, 19, 20, 21, 22, 23, 31, 30, 29, 28, 27, 26, 25, 24, 16, 8]
POS = [0] * N_DEV
for _p, _d in enumerate(RING):
    POS[_d] = _p
RIGHT = [RING[(POS[d] + 1) % N_DEV] for d in range(N_DEV)]
LEFT = [RING[(POS[d] - 1) % N_DEV] for d in range(N_DEV)]


def kernel(x, w_mat):
    m, k_local = x.shape
    _, n = w_mat.shape
    chunk = m // N_DEV
    rsub = chunk // N_STREAM

    def body(x_ref, w_ref, tbl_ref, out_ref, send_buf, recv_buf,
             send_sems, recv_sems, store_sems, *credit_sems):
        me = lax.axis_index("i")

        lane = lax.broadcasted_iota(jnp.int32, (1, 128), 1)

        def lut(row):
            return jnp.sum(jnp.where(lane == me, tbl_ref[row:row + 1, :], 0))

        r = lut(0)
        right_id = lut(1)
        left_id = lut(2)
        rb = lax.rem(N_DEV - r, N_DEV)

        pos = [r] * N_SUB + [rb] * N_SUB
        own = [lax.rem(p + 1, N_DEV) for p in pos]
        send_peer = [right_id] * N_SUB + [left_id] * N_SUB
        credit_peer = [left_id] * N_SUB + [right_id] * N_SUB
        roff = [j * rsub for j in range(N_STREAM)]

        barrier_sem = pltpu.get_barrier_semaphore()
        pl.semaphore_signal(barrier_sem, inc=1, device_id=(left_id,),
                            device_id_type=pl.DeviceIdType.MESH)
        pl.semaphore_signal(barrier_sem, inc=1, device_id=(right_id,),
                            device_id_type=pl.DeviceIdType.MESH)
        pl.semaphore_wait(barrier_sem, 2)

        out_ref[:, :] = jnp.dot(
            x_ref[:, :].astype(jnp.bfloat16),
            w_ref[:, :].astype(jnp.bfloat16),
            preferred_element_type=jnp.float32,
        ).astype(jnp.bfloat16)

        def oslice(j, c):
            return (pl.ds(c * chunk + roff[j], rsub), slice(None))

        def start_send(j, src, slot4, slot2):
            pltpu.make_async_remote_copy(
                src_ref=src,
                dst_ref=recv_buf.at[j, slot4],
                send_sem=send_sems.at[j, slot2],
                recv_sem=recv_sems.at[j, slot4],
                device_id=(send_peer[j],),
                device_id_type=pl.DeviceIdType.MESH,
            ).start()

        def wait_send_slot(j, slot2):
            pltpu.make_async_remote_copy(
                src_ref=send_buf.at[j, 0],
                dst_ref=recv_buf.at[j, 0],
                send_sem=send_sems.at[j, slot2],
                recv_sem=recv_sems.at[j, 0],
                device_id=(send_peer[j],),
                device_id_type=pl.DeviceIdType.MESH,
            ).wait_send()

        def wait_recv_slot(j, slot4):
            pltpu.make_async_remote_copy(
                src_ref=send_buf.at[j, 0],
                dst_ref=recv_buf.at[j, slot4],
                send_sem=send_sems.at[j, 0],
                recv_sem=recv_sems.at[j, slot4],
                device_id=(send_peer[j],),
                device_id_type=pl.DeviceIdType.MESH,
            ).wait_recv()

        def send_credit(j):
            pl.semaphore_signal(credit_sems[j], inc=1,
                                device_id=(credit_peer[j],),
                                device_id_type=pl.DeviceIdType.MESH)

        def start_store(j, slot4, c):
            pltpu.make_async_copy(
                recv_buf.at[j, slot4],
                out_ref.at[oslice(j, c)],
                store_sems.at[j],
            ).start()

        def wait_store(j):
            pltpu.make_async_copy(
                recv_buf.at[j, 0], out_ref.at[oslice(j, 0)],
                store_sems.at[j],
            ).wait()

        for j in range(N_STREAM):
            send_buf[j, 0] = out_ref[oslice(j, pos[j])]
            start_send(j, send_buf.at[j, 0], 0, 0)

        def rs_step(s, k, credit_wait, wait_prev, last):
            slot4 = k % 4
            nk4 = (k + 1) % 4
            nk2 = (k + 1) % 2
            for j in range(N_STREAM):
                wait_recv_slot(j, slot4)
                if wait_prev:
                    wait_send_slot(j, nk2)
                dst = lax.rem(pos[j] - s - 1 + 2 * N_DEV, N_DEV)
                if not last:
                    send_buf[j, nk2] = (
                        out_ref[oslice(j, dst)].astype(jnp.float32)
                        + recv_buf[j, slot4].astype(jnp.float32)
                    ).astype(jnp.bfloat16)
                else:
                    full = (out_ref[oslice(j, dst)].astype(jnp.float32)
                            + recv_buf[j, slot4].astype(jnp.float32))
                    yb = (full * jax.nn.sigmoid(full)).astype(jnp.bfloat16)
                    out_ref[oslice(j, dst)] = yb
                    send_buf[j, nk2] = yb
                if credit_wait:
                    pl.semaphore_wait(credit_sems[j], 1)
                start_send(j, send_buf.at[j, nk2], nk4, nk2)
                send_credit(j)

        rs_step(0, 0, False, False, False)
        rs_step(1, 1, False, True, False)
        rs_step(2, 2, False, True, False)
        rs_step(3, 3, True, True, False)

        def rs_block(b, carry):
            s = 4 * b
            for k in range(4):
                rs_step(s + k, k, True, True, False)
            return carry

        lax.fori_loop(1, 7, rs_block, 0)

        rs_step(28, 0, True, True, False)
        rs_step(29, 1, True, True, False)
        rs_step(30, 2, True, True, True)

        def ag_step(t, k4, nk4, k2, forward, wait_st, signal, guard):
            for j in range(N_STREAM):
                wait_recv_slot(j, k4)
                wait_send_slot(j, k2)
                if forward:
                    pl.semaphore_wait(credit_sems[j], 1)
                    start_send(j, recv_buf.at[j, k4], nk4, k2)
                if wait_st:
                    wait_store(j)
                c = lax.rem(own[j] - t - 1 + 2 * N_DEV, N_DEV)
                start_store(j, k4, c)
                if signal:
                    if guard:
                        @pl.when(t <= 28)
                        def _(j=j):
                            send_credit(j)
                    else:
                        send_credit(j)

        ag_step(0, 3, 0, 0, True, False, False, False)
        ag_step(1, 0, 1, 1, True, True, False, False)

        def ag_block(b, carry):
            t0 = 2 + 4 * b
            for kk in range(4):
                ag_step(t0 + kk, (1 + kk) % 4, (2 + kk) % 4, kk % 2,
                        True, True, True, True)
            return carry

        lax.fori_loop(0, 7, ag_block, 0)

        ag_step(30, 1, None, 0, False, True, False, False)

        for j in range(N_STREAM):
            wait_store(j)
            wait_send_slot(j, 1)

    pad = [0] * (128 - N_DEV)
    tables = jnp.array(
        [POS + pad, RIGHT + pad, LEFT + pad], dtype=jnp.int32
    )

    return pl.pallas_call(
        body,
        out_shape=jax.ShapeDtypeStruct((m, n), jnp.bfloat16),
        in_specs=[
            pl.BlockSpec(memory_space=pltpu.VMEM),
            pl.BlockSpec(memory_space=pltpu.VMEM),
            pl.BlockSpec(memory_space=pltpu.VMEM),
        ],
        out_specs=pl.BlockSpec(memory_space=pltpu.VMEM),
        scratch_shapes=[
            pltpu.VMEM((N_STREAM, 2, rsub, n), jnp.bfloat16),
            pltpu.VMEM((N_STREAM, N_SLOT, rsub, n), jnp.bfloat16),
            pltpu.SemaphoreType.DMA((N_STREAM, 2)),
            pltpu.SemaphoreType.DMA((N_STREAM, N_SLOT)),
            pltpu.SemaphoreType.DMA((N_STREAM,)),
        ] + [pltpu.SemaphoreType.REGULAR] * N_STREAM,
        compiler_params=pltpu.CompilerParams(
            collective_id=0,
            vmem_limit_bytes=100 * 1024 * 1024,
        ),
    )(x, w_mat, tables)


# device time: 219199 ns/iter; 3.8064x vs baseline; 1.0130x over previous
import jax
import jax.numpy as jnp
from jax import lax
from jax.experimental import pallas as pl
from jax.experimental.pallas import tpu as pltpu

N_DEV = 32
N_SUB = 4
N_STREAM = 2 * N_SUB
N_SLOT = 4

RING = [0, 1, 2, 3, 4, 5, 6, 7, 15, 14, 13, 12, 11, 10, 9, 17,
        18, 19, 20, 21, 22, 23, 31, 30, 29, 28, 27, 26, 25, 24, 16, 8]
POS = [0] * N_DEV
for _p, _d in enumerate(RING):
    POS[_d] = _p
RIGHT = [RING[(POS[d] + 1) % N_DEV] for d in range(N_DEV)]
LEFT = [RING[(POS[d] - 1) % N_DEV] for d in range(N_DEV)]


def kernel(x, w_mat):
    m, k_local = x.shape
    _, n = w_mat.shape
    chunk = m // N_DEV
    rsub = chunk // N_STREAM

    def body(x_ref, w_ref, tbl_ref, out_ref, send_buf, recv_buf,
             send_sems, recv_sems, store_sems, *credit_sems):
        me = lax.axis_index("i")

        lane = lax.broadcasted_iota(jnp.int32, (1, 128), 1)

        def lut(row):
            return jnp.sum(jnp.where(lane == me, tbl_ref[row:row + 1, :], 0))

        r = lut(0)
        right_id = lut(1)
        left_id = lut(2)
        rb = lax.rem(N_DEV - r, N_DEV)

        pos = [r] * N_SUB + [rb] * N_SUB
        own = [lax.rem(p + 1, N_DEV) for p in pos]
        send_peer = [right_id] * N_SUB + [left_id] * N_SUB
        credit_peer = [left_id] * N_SUB + [right_id] * N_SUB
        roff = [j * rsub for j in range(N_STREAM)]

        barrier_sem = pltpu.get_barrier_semaphore()
        pl.semaphore_signal(barrier_sem, inc=1, device_id=(left_id,),
                            device_id_type=pl.DeviceIdType.MESH)
        pl.semaphore_signal(barrier_sem, inc=1, device_id=(right_id,),
                            device_id_type=pl.DeviceIdType.MESH)
        pl.semaphore_wait(barrier_sem, 2)

        wb = w_ref[:, :].astype(jnp.bfloat16)

        def gemm_rows(row_ds):
            out_ref[row_ds, :] = jnp.dot(
                x_ref[row_ds, :].astype(jnp.bfloat16), wb,
                preferred_element_type=jnp.float32,
            ).astype(jnp.bfloat16)

        def oslice(j, c):
            return (pl.ds(c * chunk + roff[j], rsub), slice(None))

        def start_send(j, src, slot4, slot2):
            pltpu.make_async_remote_copy(
                src_ref=src,
                dst_ref=recv_buf.at[j, slot4],
                send_sem=send_sems.at[j, slot2],
                recv_sem=recv_sems.at[j, slot4],
                device_id=(send_peer[j],),
                device_id_type=pl.DeviceIdType.MESH,
            ).start()

        def wait_send_slot(j, slot2):
            pltpu.make_async_remote_copy(
                src_ref=send_buf.at[j, 0],
                dst_ref=recv_buf.at[j, 0],
                send_sem=send_sems.at[j, slot2],
                recv_sem=recv_sems.at[j, 0],
                device_id=(send_peer[j],),
                device_id_type=pl.DeviceIdType.MESH,
            ).wait_send()

        def wait_recv_slot(j, slot4):
            pltpu.make_async_remote_copy(
                src_ref=send_buf.at[j, 0],
                dst_ref=recv_buf.at[j, slot4],
                send_sem=send_sems.at[j, 0],
                recv_sem=recv_sems.at[j, slot4],
                device_id=(send_peer[j],),
                device_id_type=pl.DeviceIdType.MESH,
            ).wait_recv()

        def send_credit(j):
            pl.semaphore_signal(credit_sems[j], inc=1,
                                device_id=(credit_peer[j],),
                                device_id_type=pl.DeviceIdType.MESH)

        def start_store(j, slot4, c):
            pltpu.make_async_copy(
                recv_buf.at[j, slot4],
                out_ref.at[oslice(j, c)],
                store_sems.at[j],
            ).start()

        def wait_store(j):
            pltpu.make_async_copy(
                recv_buf.at[j, 0], out_ref.at[oslice(j, 0)],
                store_sems.at[j],
            ).wait()

        gemm_rows(pl.ds(r * chunk, chunk))
        gemm_rows(pl.ds(rb * chunk, chunk))
        for j in range(N_STREAM):
            send_buf[j, 0] = out_ref[oslice(j, pos[j])]
            start_send(j, send_buf.at[j, 0], 0, 0)
        gemm_rows(pl.ds(0, m))

        def rs_step(s, k, credit_wait, wait_prev, last):
            slot4 = k % 4
            nk4 = (k + 1) % 4
            nk2 = (k + 1) % 2
            for j in range(N_STREAM):
                wait_recv_slot(j, slot4)
                if wait_prev:
                    wait_send_slot(j, nk2)
                dst = lax.rem(pos[j] - s - 1 + 2 * N_DEV, N_DEV)
                if not last:
                    send_buf[j, nk2] = (
                        out_ref[oslice(j, dst)] + recv_buf[j, slot4]
                    )
                else:
                    full = (out_ref[oslice(j, dst)].astype(jnp.float32)
                            + recv_buf[j, slot4].astype(jnp.float32))
                    yb = (full * jax.nn.sigmoid(full)).astype(jnp.bfloat16)
                    out_ref[oslice(j, dst)] = yb
                    send_buf[j, nk2] = yb
                if credit_wait:
                    pl.semaphore_wait(credit_sems[j], 1)
                start_send(j, send_buf.at[j, nk2], nk4, nk2)
                send_credit(j)

        rs_step(0, 0, False, False, False)
        rs_step(1, 1, False, True, False)
        rs_step(2, 2, False, True, False)
        rs_step(3, 3, True, True, False)

        def rs_block(b, carry):
            s = 4 * b
            for k in range(4):
                rs_step(s + k, k, True, True, False)
            return carry

        lax.fori_loop(1, 7, rs_block, 0)

        rs_step(28, 0, True, True, False)
        rs_step(29, 1, True, True, False)
        rs_step(30, 2, True, True, True)

        def ag_step(t, k4, nk4, k2, forward, wait_st, signal, guard):
            for j in range(N_STREAM):
                wait_recv_slot(j, k4)
                wait_send_slot(j, k2)
                if forward:
                    pl.semaphore_wait(credit_sems[j], 1)
                    start_send(j, recv_buf.at[j, k4], nk4, k2)
                if wait_st:
                    wait_store(j)
                c = lax.rem(own[j] - t - 1 + 2 * N_DEV, N_DEV)
                start_store(j, k4, c)
                if signal:
                    if guard:
                        @pl.when(t <= 28)
                        def _(j=j):
                            send_credit(j)
                    else:
                        send_credit(j)

        ag_step(0, 3, 0, 0, True, False, False, False)
        ag_step(1, 0, 1, 1, True, True, False, False)

        def ag_block(b, carry):
            t0 = 2 + 4 * b
            for kk in range(4):
                ag_step(t0 + kk, (1 + kk) % 4, (2 + kk) % 4, kk % 2,
                        True, True, True, True)
            return carry

        lax.fori_loop(0, 7, ag_block, 0)

        ag_step(30, 1, None, 0, False, True, False, False)

        for j in range(N_STREAM):
            wait_store(j)
            wait_send_slot(j, 1)

    pad = [0] * (128 - N_DEV)
    tables = jnp.array(
        [POS + pad, RIGHT + pad, LEFT + pad], dtype=jnp.int32
    )

    return pl.pallas_call(
        body,
        out_shape=jax.ShapeDtypeStruct((m, n), jnp.bfloat16),
        in_specs=[
            pl.BlockSpec(memory_space=pltpu.VMEM),
            pl.BlockSpec(memory_space=pltpu.VMEM),
            pl.BlockSpec(memory_space=pltpu.VMEM),
        ],
        out_specs=pl.BlockSpec(memory_space=pltpu.VMEM),
        scratch_shapes=[
            pltpu.VMEM((N_STREAM, 2, rsub, n), jnp.bfloat16),
            pltpu.VMEM((N_STREAM, N_SLOT, rsub, n), jnp.bfloat16),
            pltpu.SemaphoreType.DMA((N_STREAM, 2)),
            pltpu.SemaphoreType.DMA((N_STREAM, N_SLOT)),
            pltpu.SemaphoreType.DMA((N_STREAM,)),
        ] + [pltpu.SemaphoreType.REGULAR] * N_STREAM,
        compiler_params=pltpu.CompilerParams(
            collective_id=0,
            vmem_limit_bytes=100 * 1024 * 1024,
        ),
    )(x, w_mat, tables)
